# baseline (speedup 1.0000x reference)
import functools

import jax
import jax.numpy as jnp
from jax import lax
from jax.experimental import pallas as pl
from jax.experimental.pallas import tpu as pltpu
from jax.experimental.pallas import tpu_sc as plsc

N = 10000
D = 128
E = 320000

NC = 2
NS = 16
CH = 128
G = 8
NG = 10
NCHUNK = NG * G
EPAD = NC * NS * NCHUNK * CH
DUMMY = N
ZROWS = ((N + 1 + NS - 1) // NS + 7) // 8 * 8
N_ACC = ZROWS * NS


def _sc_segment_sum(features, src_r, dst_r):
    mesh = plsc.VectorSubcoreMesh(core_axis_name="c", subcore_axis_name="s")

    @functools.partial(
        pl.kernel,
        out_type=jax.ShapeDtypeStruct((NC, N_ACC, D), jnp.float32),
        mesh=mesh,
        scratch_types=[
            pltpu.VMEM((G, CH), jnp.int32),
            pltpu.VMEM((G, CH), jnp.int32),
            pltpu.VMEM((G, CH), jnp.int32),
            pltpu.VMEM((G, CH), jnp.int32),
            pltpu.VMEM((1, CH), jnp.int32),
            pltpu.VMEM((CH, D), jnp.float32),
            pltpu.VMEM((CH, D), jnp.float32),
            pltpu.VMEM_SHARED((N_ACC, D), jnp.float32),
            pltpu.SemaphoreType.DMA,
            pltpu.SemaphoreType.DMA,
            pltpu.SemaphoreType.DMA,
            pltpu.SemaphoreType.DMA,
            pltpu.SemaphoreType.DMA,
        ],
    )
    def seg_sum(feat_hbm, srci_hbm, dsti_hbm, part_hbm,
                srcA, dstA, srcB, dstB, dumv, rows0, rows1, acc,
                sem_g0, sem_g1, sem_s0, sem_s1, sem_i):
        c = lax.axis_index("c")
        s = lax.axis_index("s")

        def drain_rows(sem, buf):
            pltpu.make_async_copy(feat_hbm.at[pl.ds(0, CH)], buf, sem).wait()

        def drain_idx():
            pltpu.make_async_copy(srci_hbm.at[0, 0, 0], srcA, sem_i).wait()
            pltpu.make_async_copy(srci_hbm.at[0, 0, 0], dstA, sem_i).wait()

        pltpu.sync_copy(srci_hbm.at[c, s, 0], srcA)
        pltpu.sync_copy(dsti_hbm.at[c, s, 0], dstA)

        for j in range(CH // 16):
            dumv[0, pl.ds(j * 16, 16)] = jnp.full((16,), DUMMY, jnp.int32)

        def zero_rows(i, carry):
            r = i // (D // 16)
            col = (i % (D // 16)) * 16
            rows0[r, pl.ds(col, 16)] = jnp.zeros((16,), jnp.float32)
            return carry

        lax.fori_loop(0, CH * (D // 16), zero_rows, 0)

        base = s * ZROWS
        nfull = ZROWS // CH
        rem = ZROWS % CH

        def zero_acc(k, carry):
            pltpu.sync_copy(rows0, acc.at[pl.ds(base + k * CH, CH)])
            return carry

        lax.fori_loop(0, nfull, zero_acc, 0)
        if rem:
            pltpu.sync_copy(rows0.at[pl.ds(0, rem)],
                            acc.at[pl.ds(base + nfull * CH, rem)])
        plsc.subcore_barrier()

        pltpu.async_copy(feat_hbm.at[srcA.at[0]], rows0, sem_g0)
        pltpu.async_copy(rows1, acc.at[dumv.at[0]], sem_s1, add=True)

        def src_row(local):
            buf = (srcA, srcB, srcA)[local // G]
            return buf.at[local % G]

        def dst_row(local):
            buf = (dstA, dstB, dstA)[local // G]
            return buf.at[local % G]

        def body(k, carry):
            g2k = 2 * k
            for p in range(G):
                a = 2 * p
                drain_rows(sem_s1, rows1)
                if p == 0:
                    pltpu.async_copy(srci_hbm.at[c, s, g2k + 1], srcB, sem_i)
                    pltpu.async_copy(dsti_hbm.at[c, s, g2k + 1], dstB, sem_i)
                if p == 4:
                    pltpu.async_copy(srci_hbm.at[c, s, g2k + 2], srcA, sem_i)
                    pltpu.async_copy(dsti_hbm.at[c, s, g2k + 2], dstA, sem_i)
                pltpu.async_copy(feat_hbm.at[src_row(a + 1)], rows1, sem_g1)
                drain_rows(sem_g0, rows0)
                pltpu.async_copy(rows0, acc.at[dst_row(a)], sem_s0, add=True)
                drain_rows(sem_s0, rows0)
                if p == 3 or p == G - 1:
                    drain_idx()
                pltpu.async_copy(feat_hbm.at[src_row(a + 2)], rows0, sem_g0)
                drain_rows(sem_g1, rows1)
                pltpu.async_copy(rows1, acc.at[dst_row(a + 1)], sem_s1,
                                 add=True)
            return carry

        lax.fori_loop(0, NG // 2, body, 0)
        drain_rows(sem_s1, rows1)
        drain_rows(sem_g0, rows0)
        plsc.subcore_barrier()

        pltpu.sync_copy(acc.at[pl.ds(base, ZROWS)],
                        part_hbm.at[c, pl.ds(base, ZROWS)])

    return seg_sum(features, src_r, dst_r)


def _tc_tail(features, p0, p1, d_norm, W0, b0, W1, b1):
    R = 2000
    grid = (N // R,)

    def body(x_ref, p0_ref, p1_ref, d_ref, w0_ref, b0_ref, w1_ref, b1_ref, o_ref):
        x = x_ref[...]
        o_ref[:, :D] = (
            jnp.dot(x, w0_ref[...], preferred_element_type=jnp.float32)
            + b0_ref[...]
        )
        h = (p0_ref[...] + p1_ref[...]) * d_ref[...]
        o_ref[:, D:] = (
            jnp.dot(h, w1_ref[...], preferred_element_type=jnp.float32)
            + b1_ref[...]
        )

    return pl.pallas_call(
        body,
        grid=grid,
        in_specs=[
            pl.BlockSpec((R, D), lambda i: (i, 0)),
            pl.BlockSpec((R, D), lambda i: (i, 0)),
            pl.BlockSpec((R, D), lambda i: (i, 0)),
            pl.BlockSpec((R, 1), lambda i: (i, 0)),
            pl.BlockSpec((D, D), lambda i: (0, 0)),
            pl.BlockSpec((1, D), lambda i: (0, 0)),
            pl.BlockSpec((D, D), lambda i: (0, 0)),
            pl.BlockSpec((1, D), lambda i: (0, 0)),
        ],
        out_specs=pl.BlockSpec((R, 2 * D), lambda i: (i, 0)),
        out_shape=jax.ShapeDtypeStruct((N, 2 * D), jnp.float32),
    )(features, p0, p1, d_norm, W0, b0, W1, b1)


def kernel(features, edge_index, D_norm, W0, b0, W1, b1):
    src = edge_index[0].astype(jnp.int32)
    dst = edge_index[1].astype(jnp.int32)
    pad = EPAD - E
    src_r = jnp.concatenate([src, jnp.zeros((pad,), jnp.int32)])
    dst_r = jnp.concatenate([dst, jnp.full((pad,), DUMMY, jnp.int32)])
    src_r = src_r.reshape(NC, NS, NG, G, CH)
    dst_r = dst_r.reshape(NC, NS, NG, G, CH)
    src_r = jnp.concatenate(
        [src_r, jnp.zeros((NC, NS, 1, G, CH), jnp.int32)], axis=2)
    dst_r = jnp.concatenate(
        [dst_r, jnp.full((NC, NS, 1, G, CH), DUMMY, jnp.int32)], axis=2)

    part = _sc_segment_sum(features, src_r, dst_r)
    return _tc_tail(features, part[0], part[1], D_norm,
                    W0, b0.reshape(1, D), W1, b1.reshape(1, D))


# speedup vs baseline: 1.2607x; 1.2607x over previous
import functools

import jax
import jax.numpy as jnp
from jax import lax
from jax.experimental import pallas as pl
from jax.experimental.pallas import tpu as pltpu
from jax.experimental.pallas import tpu_sc as plsc

N = 10000
D = 128
E = 320000

NC = 2
NS = 16
CH = 128
G = 8
NG = 10
NCHUNK = NG * G
EPAD = NC * NS * NCHUNK * CH
DUMMY = N
ZROWS = ((N + 1 + NS - 1) // NS + 7) // 8 * 8
N_ACC = ZROWS * NS


def _sc_segment_sum(features, src_r, dst_r):
    mesh = plsc.VectorSubcoreMesh(core_axis_name="c", subcore_axis_name="s")

    @functools.partial(
        pl.kernel,
        out_type=jax.ShapeDtypeStruct((NC, N_ACC, D), jnp.float32),
        mesh=mesh,
        scratch_types=[
            pltpu.VMEM((G, CH), jnp.int32),
            pltpu.VMEM((G, CH), jnp.int32),
            pltpu.VMEM((1, CH), jnp.int32),
            pltpu.VMEM((CH, D), jnp.float32),
            pltpu.VMEM((CH, D), jnp.float32),
            pltpu.VMEM_SHARED((N_ACC, D), jnp.float32),
            pltpu.SemaphoreType.DMA,
            pltpu.SemaphoreType.DMA,
            pltpu.SemaphoreType.DMA,
        ],
    )
    def seg_sum(feat_hbm, srci_hbm, dsti_hbm, part_hbm,
                srcA, dstA, dumv, rows0, rows1, acc, sem_g, sem_s0, sem_s1):
        c = lax.axis_index("c")
        s = lax.axis_index("s")
        base = s * ZROWS

        def drain_rows(sem, buf):
            pltpu.make_async_copy(feat_hbm.at[pl.ds(0, CH)], buf, sem).wait()

        for j in range(CH // 16):
            dumv[0, pl.ds(j * 16, 16)] = jnp.full((16,), DUMMY, jnp.int32)

        def zero_rows(i, carry):
            r = i // (D // 16)
            col = (i % (D // 16)) * 16
            rows0[r, pl.ds(col, 16)] = jnp.zeros((16,), jnp.float32)
            return carry

        lax.fori_loop(0, CH * (D // 16), zero_rows, 0)

        nfull = ZROWS // CH
        rem = ZROWS % CH

        def zero_acc(k, carry):
            pltpu.sync_copy(rows0, acc.at[pl.ds(base + k * CH, CH)])
            return carry

        lax.fori_loop(0, nfull, zero_acc, 0)
        if rem:
            pltpu.sync_copy(rows0.at[pl.ds(0, rem)],
                            acc.at[pl.ds(base + nfull * CH, rem)])
        plsc.subcore_barrier()

        pltpu.async_copy(rows0, acc.at[dumv.at[0]], sem_s0, add=True)
        pltpu.async_copy(rows1, acc.at[dumv.at[0]], sem_s1, add=True)

        def body(g, carry):
            pltpu.sync_copy(srci_hbm.at[c, s, g], srcA)
            pltpu.sync_copy(dsti_hbm.at[c, s, g], dstA)
            for b in range(G):
                rows = rows0 if b % 2 == 0 else rows1
                sem_s = sem_s0 if b % 2 == 0 else sem_s1
                drain_rows(sem_s, rows)
                pltpu.async_copy(feat_hbm.at[srcA.at[b]], rows, sem_g).wait()
                pltpu.async_copy(rows, acc.at[dstA.at[b]], sem_s, add=True)
            return carry

        lax.fori_loop(0, NG, body, 0)
        drain_rows(sem_s0, rows0)
        drain_rows(sem_s1, rows1)
        plsc.subcore_barrier()

        pltpu.sync_copy(acc.at[pl.ds(base, ZROWS)],
                        part_hbm.at[c, pl.ds(base, ZROWS)])

    return seg_sum(features, src_r, dst_r)


def _tc_tail(features, p0, p1, d_norm, W0, b0, W1, b1):
    R = 2000
    grid = (N // R,)

    def body(x_ref, p0_ref, p1_ref, d_ref, w0_ref, b0_ref, w1_ref, b1_ref, o_ref):
        x = x_ref[...]
        o_ref[:, :D] = (
            jnp.dot(x, w0_ref[...], preferred_element_type=jnp.float32)
            + b0_ref[...]
        )
        h = (p0_ref[...] + p1_ref[...]) * d_ref[...]
        o_ref[:, D:] = (
            jnp.dot(h, w1_ref[...], preferred_element_type=jnp.float32)
            + b1_ref[...]
        )

    return pl.pallas_call(
        body,
        grid=grid,
        in_specs=[
            pl.BlockSpec((R, D), lambda i: (i, 0)),
            pl.BlockSpec((R, D), lambda i: (i, 0)),
            pl.BlockSpec((R, D), lambda i: (i, 0)),
            pl.BlockSpec((R, 1), lambda i: (i, 0)),
            pl.BlockSpec((D, D), lambda i: (0, 0)),
            pl.BlockSpec((1, D), lambda i: (0, 0)),
            pl.BlockSpec((D, D), lambda i: (0, 0)),
            pl.BlockSpec((1, D), lambda i: (0, 0)),
        ],
        out_specs=pl.BlockSpec((R, 2 * D), lambda i: (i, 0)),
        out_shape=jax.ShapeDtypeStruct((N, 2 * D), jnp.float32),
    )(features, p0, p1, d_norm, W0, b0, W1, b1)


def kernel(features, edge_index, D_norm, W0, b0, W1, b1):
    src = edge_index[0].astype(jnp.int32)
    dst = edge_index[1].astype(jnp.int32)
    pad = EPAD - E
    src_r = jnp.concatenate([src, jnp.zeros((pad,), jnp.int32)])
    dst_r = jnp.concatenate([dst, jnp.full((pad,), DUMMY, jnp.int32)])
    src_r = src_r.reshape(NC, NS, NG, G, CH)
    dst_r = dst_r.reshape(NC, NS, NG, G, CH)

    part = _sc_segment_sum(features, src_r, dst_r)
    return _tc_tail(features, part[0], part[1], D_norm,
                    W0, b0.reshape(1, D), W1, b1.reshape(1, D))


# speedup vs baseline: 1.2864x; 1.0203x over previous
import functools

import jax
import jax.numpy as jnp
from jax import lax
from jax.experimental import pallas as pl
from jax.experimental.pallas import tpu as pltpu
from jax.experimental.pallas import tpu_sc as plsc

N = 10000
D = 128
E = 320000

NC = 2
NS = 16
CH = 128
G = 8
NG = 10
NCHUNK = NG * G
EPAD = NC * NS * NCHUNK * CH
DUMMY = N
ZROWS = ((N + 1 + NS - 1) // NS + 7) // 8 * 8
N_ACC = ZROWS * NS


def _sc_segment_sum(features, src_r, dst_r):
    mesh = plsc.VectorSubcoreMesh(core_axis_name="c", subcore_axis_name="s")

    @functools.partial(
        pl.kernel,
        out_type=jax.ShapeDtypeStruct((NC, N_ACC, D), jnp.float32),
        mesh=mesh,
        scratch_types=[
            pltpu.VMEM((G, CH), jnp.int32),
            pltpu.VMEM((G, CH), jnp.int32),
            pltpu.VMEM((1, CH), jnp.int32),
            pltpu.VMEM((CH, D), jnp.float32),
            pltpu.VMEM((CH, D), jnp.float32),
            pltpu.VMEM_SHARED((N_ACC, D), jnp.float32),
            pltpu.SemaphoreType.DMA,
            pltpu.SemaphoreType.DMA,
            pltpu.SemaphoreType.DMA,
            pltpu.SemaphoreType.DMA,
        ],
    )
    def seg_sum(feat_hbm, srci_hbm, dsti_hbm, part_hbm,
                srcA, dstA, dumv, rows0, rows1, acc,
                sem_g0, sem_g1, sem_s0, sem_s1):
        c = lax.axis_index("c")
        s = lax.axis_index("s")
        base = s * ZROWS

        def drain_rows(sem, buf):
            pltpu.make_async_copy(feat_hbm.at[pl.ds(0, CH)], buf, sem).wait()

        for j in range(CH // 16):
            dumv[0, pl.ds(j * 16, 16)] = jnp.full((16,), DUMMY, jnp.int32)

        def zero_rows(i, carry):
            r = i // (D // 16)
            col = (i % (D // 16)) * 16
            rows0[r, pl.ds(col, 16)] = jnp.zeros((16,), jnp.float32)
            return carry

        lax.fori_loop(0, CH * (D // 16), zero_rows, 0)

        nfull = ZROWS // CH
        rem = ZROWS % CH

        def zero_acc(k, carry):
            pltpu.sync_copy(rows0, acc.at[pl.ds(base + k * CH, CH)])
            return carry

        lax.fori_loop(0, nfull, zero_acc, 0)
        if rem:
            pltpu.sync_copy(rows0.at[pl.ds(0, rem)],
                            acc.at[pl.ds(base + nfull * CH, rem)])
        plsc.subcore_barrier()

        def body(g, carry):
            pltpu.sync_copy(srci_hbm.at[c, s, g], srcA)
            pltpu.sync_copy(dsti_hbm.at[c, s, g], dstA)
            pltpu.async_copy(feat_hbm.at[srcA.at[0]], rows0, sem_g0)
            for b in range(G):
                if b % 2 == 0:
                    rows, sem_g, sem_s = rows0, sem_g0, sem_s0
                    rowsq, sem_gq, sem_sq = rows1, sem_g1, sem_s1
                else:
                    rows, sem_g, sem_s = rows1, sem_g1, sem_s1
                    rowsq, sem_gq, sem_sq = rows0, sem_g0, sem_s0
                if b < G - 1:
                    if b >= 1:
                        drain_rows(sem_sq, rowsq)
                    pltpu.async_copy(feat_hbm.at[srcA.at[b + 1]], rowsq,
                                     sem_gq)
                drain_rows(sem_g, rows)
                pltpu.async_copy(rows, acc.at[dstA.at[b]], sem_s, add=True)
            drain_rows(sem_s0, rows0)
            drain_rows(sem_s1, rows1)
            return carry

        lax.fori_loop(0, NG, body, 0)
        plsc.subcore_barrier()

        pltpu.sync_copy(acc.at[pl.ds(base, ZROWS)],
                        part_hbm.at[c, pl.ds(base, ZROWS)])

    return seg_sum(features, src_r, dst_r)


def _tc_tail(features, p0, p1, d_norm, W0, b0, W1, b1):
    R = 2000
    grid = (N // R,)

    def body(x_ref, p0_ref, p1_ref, d_ref, w0_ref, b0_ref, w1_ref, b1_ref, o_ref):
        x = x_ref[...]
        o_ref[:, :D] = (
            jnp.dot(x, w0_ref[...], preferred_element_type=jnp.float32)
            + b0_ref[...]
        )
        h = (p0_ref[...] + p1_ref[...]) * d_ref[...]
        o_ref[:, D:] = (
            jnp.dot(h, w1_ref[...], preferred_element_type=jnp.float32)
            + b1_ref[...]
        )

    return pl.pallas_call(
        body,
        grid=grid,
        in_specs=[
            pl.BlockSpec((R, D), lambda i: (i, 0)),
            pl.BlockSpec((R, D), lambda i: (i, 0)),
            pl.BlockSpec((R, D), lambda i: (i, 0)),
            pl.BlockSpec((R, 1), lambda i: (i, 0)),
            pl.BlockSpec((D, D), lambda i: (0, 0)),
            pl.BlockSpec((1, D), lambda i: (0, 0)),
            pl.BlockSpec((D, D), lambda i: (0, 0)),
            pl.BlockSpec((1, D), lambda i: (0, 0)),
        ],
        out_specs=pl.BlockSpec((R, 2 * D), lambda i: (i, 0)),
        out_shape=jax.ShapeDtypeStruct((N, 2 * D), jnp.float32),
    )(features, p0, p1, d_norm, W0, b0, W1, b1)


def kernel(features, edge_index, D_norm, W0, b0, W1, b1):
    src = edge_index[0].astype(jnp.int32)
    dst = edge_index[1].astype(jnp.int32)
    pad = EPAD - E
    src_r = jnp.concatenate([src, jnp.zeros((pad,), jnp.int32)])
    dst_r = jnp.concatenate([dst, jnp.full((pad,), DUMMY, jnp.int32)])
    src_r = src_r.reshape(NC, NS, NG, G, CH)
    dst_r = dst_r.reshape(NC, NS, NG, G, CH)

    part = _sc_segment_sum(features, src_r, dst_r)
    return _tc_tail(features, part[0], part[1], D_norm,
                    W0, b0.reshape(1, D), W1, b1.reshape(1, D))
